# baseline (speedup 1.0000x reference)
import functools

import jax
import jax.numpy as jnp
from jax import lax
from jax.experimental import pallas as pl
from jax.experimental.pallas import tpu as pltpu
from jax.experimental.pallas import tpu_sc as plsc

NC = 2
NS = 16
NW = NC * NS
D = 128
N = 64
S = 4


@functools.partial(jax.jit, static_argnames=("bc",))
def _run(src, pth, tgt, token_table, path_table, bc):
    per_w = bc // NW
    steps = per_w // N
    groups = steps // S
    mesh = plsc.VectorSubcoreMesh(core_axis_name="c", subcore_axis_name="s")

    @functools.partial(
        pl.kernel,
        mesh=mesh,
        out_type=jax.ShapeDtypeStruct((bc, 3 * D), jnp.float32),
        scratch_types=[
            pltpu.VMEM((per_w,), jnp.int32),
            pltpu.VMEM((per_w,), jnp.int32),
            pltpu.VMEM((per_w,), jnp.int32),
        ] + [pltpu.VMEM((N, 3 * D), jnp.float32)] * S
          + [pltpu.SemaphoreType.DMA] * (4 * S),
    )
    def k(src_hbm, pth_hbm, tgt_hbm, token_hbm, path_hbm, out_hbm,
          src_v, pth_v, tgt_v, *bufs_and_sems):
        bufs = bufs_and_sems[:S]
        gsems = [bufs_and_sems[S + 3 * s:S + 3 * s + 3] for s in range(S)]
        wsems = bufs_and_sems[4 * S:5 * S]
        wid = lax.axis_index("s") * NC + lax.axis_index("c")
        base = wid * per_w
        pltpu.sync_copy(src_hbm.at[pl.ds(base, per_w)], src_v)
        pltpu.sync_copy(pth_hbm.at[pl.ds(base, per_w)], pth_v)
        pltpu.sync_copy(tgt_hbm.at[pl.ds(base, per_w)], tgt_v)

        sets = tuple((bufs[s], gsems[s], wsems[s]) for s in range(S))

        def gfire(t, s):
            b, g, _ = sets[s]
            off = t * N
            pltpu.async_copy(token_hbm.at[src_v.at[pl.ds(off, N)]], b.at[:, pl.ds(0, D)], g[0])
            pltpu.async_copy(path_hbm.at[pth_v.at[pl.ds(off, N)]], b.at[:, pl.ds(D, D)], g[1])
            pltpu.async_copy(token_hbm.at[tgt_v.at[pl.ds(off, N)]], b.at[:, pl.ds(2 * D, D)], g[2])

        def gwait_band(s, f):
            b, g, _ = sets[s]
            pltpu.make_async_copy(out_hbm.at[pl.ds(0, N), pl.ds(0, D)], b.at[:, pl.ds(f * D, D)], g[f]).wait()

        def wfire_band(t, s, f):
            b, _, w = sets[s]
            row0 = base + t * N
            pltpu.async_copy(b.at[:, pl.ds(f * D, D)], out_hbm.at[pl.ds(row0, N), pl.ds(f * D, D)], w)

        def wwait(s):
            b, _, w = sets[s]
            for f in range(3):
                pltpu.make_async_copy(b.at[:, pl.ds(f * D, D)], out_hbm.at[pl.ds(0, N), pl.ds(0, D)], w).wait()

        gfire(0, 0)
        gfire(1, 1)

        def body(i, carry):
            for a in range(S):
                t = S * i + a
                sn = (a + 2) % S
                with jax.named_scope("drain_w"):
                    @pl.when(t >= 2)
                    def _():
                        wwait(sn)
                with jax.named_scope("next_g"):
                    @pl.when(t + 2 < steps)
                    def _():
                        gfire(t + 2, sn)
                for f in range(3):
                    gwait_band(a, f)
                    wfire_band(t, a, f)
            return carry

        lax.fori_loop(0, groups, body, 0)
        wwait((steps - 2) % S)
        wwait((steps - 1) % S)

    return k(src, pth, tgt, token_table, path_table)


def kernel(path_source_token_idxs, path_idxs, path_target_token_idxs, token_table, path_table):
    b, c = path_source_token_idxs.shape
    bc = b * c
    src = path_source_token_idxs.reshape(bc).astype(jnp.int32)
    pth = path_idxs.reshape(bc).astype(jnp.int32)
    tgt = path_target_token_idxs.reshape(bc).astype(jnp.int32)
    out = _run(src, pth, tgt, token_table, path_table, bc)
    return out.reshape(b, c, 3 * D)


# speedup vs baseline: 1.0018x; 1.0018x over previous
import functools

import jax
import jax.numpy as jnp
from jax import lax
from jax.experimental import pallas as pl
from jax.experimental.pallas import tpu as pltpu
from jax.experimental.pallas import tpu_sc as plsc

NC = 2
NS = 16
NW = NC * NS
D = 128
N = 64
S = 4


@functools.partial(jax.jit, static_argnames=("bc",))
def _run(src, pth, tgt, token_table, path_table, bc):
    per_w = bc // NW
    steps = per_w // N
    groups = steps // S
    mesh = plsc.VectorSubcoreMesh(core_axis_name="c", subcore_axis_name="s")

    @functools.partial(
        pl.kernel,
        mesh=mesh,
        out_type=jax.ShapeDtypeStruct((bc, 3 * D), jnp.float32),
        scratch_types=[
            pltpu.VMEM((per_w,), jnp.int32),
            pltpu.VMEM((per_w,), jnp.int32),
            pltpu.VMEM((per_w,), jnp.int32),
        ] + [pltpu.VMEM((N, 3 * D), jnp.float32)] * S
          + [pltpu.SemaphoreType.DMA] * (4 * S),
    )
    def k(src_hbm, pth_hbm, tgt_hbm, token_hbm, path_hbm, out_hbm,
          src_v, pth_v, tgt_v, *bufs_and_sems):
        bufs = bufs_and_sems[:S]
        gsems = [bufs_and_sems[S + 3 * s:S + 3 * s + 3] for s in range(S)]
        wsems = bufs_and_sems[4 * S:5 * S]
        wid = lax.axis_index("s") * NC + lax.axis_index("c")
        base = wid * per_w
        pltpu.sync_copy(src_hbm.at[pl.ds(base, per_w)], src_v)
        pltpu.sync_copy(pth_hbm.at[pl.ds(base, per_w)], pth_v)
        pltpu.sync_copy(tgt_hbm.at[pl.ds(base, per_w)], tgt_v)

        sets = tuple((bufs[s], gsems[s], wsems[s]) for s in range(S))

        def gfire(t, s):
            b, g, _ = sets[s]
            off = t * N
            pltpu.async_copy(token_hbm.at[src_v.at[pl.ds(off, N)]], b.at[:, pl.ds(0, D)], g[0])
            pltpu.async_copy(path_hbm.at[pth_v.at[pl.ds(off, N)]], b.at[:, pl.ds(D, D)], g[1])
            pltpu.async_copy(token_hbm.at[tgt_v.at[pl.ds(off, N)]], b.at[:, pl.ds(2 * D, D)], g[2])

        def gwait_band(s, f):
            b, g, _ = sets[s]
            pltpu.make_async_copy(out_hbm.at[pl.ds(0, N), pl.ds(0, D)], b.at[:, pl.ds(f * D, D)], g[f]).wait()

        def wfire_band(t, s, f):
            b, _, w = sets[s]
            row0 = base + t * N
            pltpu.async_copy(b.at[:, pl.ds(f * D, D)], out_hbm.at[pl.ds(row0, N), pl.ds(f * D, D)], w)

        def wwait(s):
            b, _, w = sets[s]
            for f in range(3):
                pltpu.make_async_copy(b.at[:, pl.ds(f * D, D)], out_hbm.at[pl.ds(0, N), pl.ds(0, D)], w).wait()

        gfire(0, 0)
        gfire(1, 1)

        def body(i, carry):
            for a in range(S):
                t = S * i + a
                for f in range(3):
                    gwait_band(a, f)
                    wfire_band(t, a, f)
                sn = (a + 2) % S
                with jax.named_scope("drain_w"):
                    @pl.when(t >= 2)
                    def _():
                        wwait(sn)
                with jax.named_scope("next_g"):
                    @pl.when(t + 2 < steps)
                    def _():
                        gfire(t + 2, sn)
            return carry

        lax.fori_loop(0, groups, body, 0)
        wwait((steps - 2) % S)
        wwait((steps - 1) % S)

    return k(src, pth, tgt, token_table, path_table)


def kernel(path_source_token_idxs, path_idxs, path_target_token_idxs, token_table, path_table):
    b, c = path_source_token_idxs.shape
    bc = b * c
    src = path_source_token_idxs.reshape(bc).astype(jnp.int32)
    pth = path_idxs.reshape(bc).astype(jnp.int32)
    tgt = path_target_token_idxs.reshape(bc).astype(jnp.int32)
    out = _run(src, pth, tgt, token_table, path_table, bc)
    return out.reshape(b, c, 3 * D)
